# Initial kernel scaffold; baseline (speedup 1.0000x reference)
import jax
import jax.numpy as jnp
from jax.experimental import pallas as pl


def kernel(x, Wg, bg, Wnoise, bnoise, eps):
    raise NotImplementedError("write your pallas kernel here")


# speedup vs baseline: 3.8410x; 3.8410x over previous
import functools

import jax
import jax.numpy as jnp
from jax.experimental import pallas as pl
from jax.experimental.pallas import tpu as pltpu

B, S, D, E = 4, 8192, 768, 8
N = B * S
T = 1024


def _fused_body(x_ref, wg_ref, wn_ref, bg_ref, bn_ref, eps_ref, out_ref):
    x = x_ref[...]
    gate = jax.lax.dot_general(
        x, wg_ref[...], (((1,), (0,)), ((), ())),
        preferred_element_type=jnp.float32) + bg_ref[...]
    noise = jax.lax.dot_general(
        x, wn_ref[...], (((1,), (0,)), ((), ())),
        preferred_element_type=jnp.float32) + bn_ref[...]
    h = gate + eps_ref[...] * jax.nn.softplus(noise)

    e_iota = jax.lax.broadcasted_iota(jnp.int32, h.shape, 1)
    m1 = jnp.max(h, axis=1, keepdims=True)
    i1 = jnp.min(jnp.where(h == m1, e_iota, E), axis=1, keepdims=True)
    h2 = jnp.where(e_iota == i1, -jnp.inf, h)
    m2 = jnp.max(h2, axis=1, keepdims=True)
    i2 = jnp.min(jnp.where(h2 == m2, e_iota, E), axis=1, keepdims=True)
    w2 = jnp.exp(m2 - m1)
    recip = 1.0 / (1.0 + w2)
    out_ref[...] = jnp.where(
        e_iota == i1, recip, jnp.where(e_iota == i2, w2 * recip, 0.0))


@jax.jit
def _gating(x2, wg_t, wn_t, bg2, bn2, eps2):
    grid = (N // T,)
    return pl.pallas_call(
        _fused_body,
        grid=grid,
        in_specs=[
            pl.BlockSpec((T, D), lambda i: (i, 0)),
            pl.BlockSpec((D, E), lambda i: (0, 0)),
            pl.BlockSpec((D, E), lambda i: (0, 0)),
            pl.BlockSpec((1, E), lambda i: (0, 0)),
            pl.BlockSpec((1, E), lambda i: (0, 0)),
            pl.BlockSpec((T, E), lambda i: (i, 0)),
        ],
        out_specs=pl.BlockSpec((T, E), lambda i: (i, 0)),
        out_shape=jax.ShapeDtypeStruct((N, E), jnp.float32),
        compiler_params=pltpu.CompilerParams(
            dimension_semantics=("arbitrary",)),
    )(x2, wg_t, wn_t, bg2, bn2, eps2)


def kernel(x, Wg, bg, Wnoise, bnoise, eps):
    x2 = x.reshape(N, D)
    eps2 = eps.reshape(N, E)
    g = _gating(x2, Wg.T, Wnoise.T, bg.reshape(1, E), bnoise.reshape(1, E),
                eps2)
    return g.reshape(B, S, E)
